# Initial kernel scaffold; baseline (speedup 1.0000x reference)
import jax
import jax.numpy as jnp
from jax.experimental import pallas as pl


def kernel(inputs, codebook):
    raise NotImplementedError("write your pallas kernel here")


# speedup vs baseline: 6.1052x; 6.1052x over previous
import jax
import jax.numpy as jnp
from jax.experimental import pallas as pl
from jax.experimental.pallas import tpu as pltpu

_N = 8192
_D = 32
_M = 8192
_R = 256
_COMMITMENT_COST = 0.25
_KLD_SCALE = 100.0


def _vq_main_body(x_ref, rn_ref, cn_ref, c_ref,
                  enc_ref, i_ref, dm_ref, q_ref, dscr):
    mm = jax.lax.dot_general(
        x_ref[...], c_ref[...], (((1,), (1,)), ((), ())),
        preferred_element_type=jnp.float32)
    dscr[...] = (rn_ref[...] + cn_ref[...]) - 2.0 * mm
    dd = dscr[...]
    dmin = jnp.min(dd, axis=1, keepdims=True)
    dm_ref[...] = dmin
    iota = jax.lax.broadcasted_iota(jnp.int32, (_R, _N), 1)
    cand = jnp.where(dd == dmin, iota, jnp.int32(_N))
    idx = jnp.min(cand, axis=1, keepdims=True)
    i_ref[...] = idx
    enc = (iota == idx).astype(jnp.float32)
    enc_ref[...] = enc
    q = jax.lax.dot_general(
        enc, c_ref[...], (((1,), (0,)), ((), ())),
        preferred_element_type=jnp.float32)
    q_ref[...] = x_ref[...] + (q - x_ref[...])


_vq_main = pl.pallas_call(
    _vq_main_body,
    grid=(_M // _R,),
    in_specs=[
        pl.BlockSpec((_R, _D), lambda i: (i, 0)),
        pl.BlockSpec((_R, 1), lambda i: (i, 0)),
        pl.BlockSpec((1, _N), lambda i: (0, 0)),
        pl.BlockSpec((_N, _D), lambda i: (0, 0)),
    ],
    out_specs=[
        pl.BlockSpec((_R, _N), lambda i: (i, 0)),
        pl.BlockSpec((_R, 1), lambda i: (i, 0)),
        pl.BlockSpec((_R, 1), lambda i: (i, 0)),
        pl.BlockSpec((_R, _D), lambda i: (i, 0)),
    ],
    out_shape=[
        jax.ShapeDtypeStruct((_M, _N), jnp.float32),
        jax.ShapeDtypeStruct((_M, 1), jnp.int32),
        jax.ShapeDtypeStruct((_M, 1), jnp.float32),
        jax.ShapeDtypeStruct((_M, _D), jnp.float32),
    ],
    scratch_shapes=[pltpu.VMEM((_R, _N), jnp.float32)],
)


def _vq_scalar_body(q_ref, x_ref, i_ref, loss_ref, ppl_ref,
                    counts_ref, lsum_ref):
    i = pl.program_id(0)
    q = q_ref[...]
    x = x_ref[...]
    iota = jax.lax.broadcasted_iota(jnp.int32, (_R, _N), 1)
    onehot = (iota == i_ref[...]).astype(jnp.float32)
    colsum = jnp.sum(onehot, axis=0, keepdims=True)
    lpart = jnp.sum((q - x) ** 2)

    @pl.when(i == 0)
    def _init():
        counts_ref[...] = colsum
        lsum_ref[0] = lpart

    @pl.when(i > 0)
    def _acc():
        counts_ref[...] += colsum
        lsum_ref[0] += lpart

    @pl.when(i == pl.num_programs(0) - 1)
    def _fin():
        m = lsum_ref[0] / jnp.float32(_M * _D)
        loss_ref[0, 0] = (m + _COMMITMENT_COST * m) * _KLD_SCALE
        avg = counts_ref[...] / jnp.float32(_M)
        ent = jnp.sum(avg * jnp.log(avg + 1e-10))
        ppl_ref[0, 0] = jnp.exp(-ent)


_vq_scalar = pl.pallas_call(
    _vq_scalar_body,
    grid=(_M // _R,),
    in_specs=[
        pl.BlockSpec((_R, _D), lambda i: (i, 0)),
        pl.BlockSpec((_R, _D), lambda i: (i, 0)),
        pl.BlockSpec((_R, 1), lambda i: (i, 0)),
    ],
    out_specs=[
        pl.BlockSpec((1, 1), lambda i: (0, 0), memory_space=pltpu.SMEM),
        pl.BlockSpec((1, 1), lambda i: (0, 0), memory_space=pltpu.SMEM),
    ],
    out_shape=[
        jax.ShapeDtypeStruct((1, 1), jnp.float32),
        jax.ShapeDtypeStruct((1, 1), jnp.float32),
    ],
    scratch_shapes=[
        pltpu.VMEM((1, _N), jnp.float32),
        pltpu.SMEM((1,), jnp.float32),
    ],
)


def kernel(inputs, codebook):
    b, ch, sl = inputs.shape
    x = jnp.transpose(inputs, (0, 2, 1)).reshape(-1, _D)
    rn = jnp.sum(x ** 2, axis=1, keepdims=True)
    cn = jnp.sum(codebook ** 2, axis=1)[None, :]
    enc, idx, _, qst = _vq_main(x, rn, cn, codebook)
    loss, ppl = _vq_scalar(qst, x, idx)
    quantized_out = jnp.transpose(qst.reshape(b, sl, ch), (0, 2, 1))
    return (loss[0, 0], quantized_out, ppl[0, 0], enc)
